# Initial kernel scaffold; baseline (speedup 1.0000x reference)
import jax
import jax.numpy as jnp
from jax.experimental import pallas as pl


def kernel(x, edge_index, edge_attr, W, We, be, bias):
    raise NotImplementedError("write your pallas kernel here")


# speedup vs baseline: 14.3989x; 14.3989x over previous
import jax
import jax.numpy as jnp
from jax import lax
from jax.experimental import pallas as pl
from jax.experimental.pallas import tpu as pltpu
from jax.experimental.pallas import tpu_sc as plsc

F32 = jnp.float32
I32 = jnp.int32

NC = 2
NS = 16
L = 16
NW = NC * NS
SUB = 128

_SC_PARAMS = dict(
    compiler_params=pltpu.CompilerParams(
        needs_layout_passes=False, use_tc_tiling_on_sc=False),
)


def _sc_mesh():
    return plsc.VectorSubcoreMesh(core_axis_name="c", subcore_axis_name="s")


def _make_pass1(E, n, n_pad):
    CHUNK = 512
    NSUB = CHUNK // SUB
    GP = CHUNK // L
    n_chunks = E // CHUNK
    max_chunks = -(-n_chunks // NW)
    zrows = n_pad // NS

    def body(rows_h, cols_h, attr_h,
             sa_h, sl_h, cnt_h,
             r_v, c_v, a_v, ia_v, ib_v,
             cnt_v, slc_v, cra_v, z_v,
             sa_s, sl_s):
        cid = lax.axis_index("c")
        sid = lax.axis_index("s")
        wid = cid * NS + sid
        zero16 = jnp.zeros((L,), F32)
        ones16 = jnp.ones((L,), F32)
        trash16 = jnp.full((L,), n, I32)

        def _zz(i, carry):
            z_v[i, :] = zero16
            return carry
        lax.fori_loop(0, 64, _zz, 0)

        def _zc(i, carry):
            cnt_v[pl.ds(i * L, L)] = zero16
            slc_v[pl.ds(i * L, L)] = zero16
            cra_v[pl.ds(i * L, L)] = zero16
            return carry
        lax.fori_loop(0, n_pad // L, _zc, 0)

        def _zs(i, carry):
            r0 = sid * zrows + i * 64
            pltpu.sync_copy(z_v, sa_s.at[pl.ds(r0, 64)])
            pltpu.sync_copy(z_v, sl_s.at[pl.ds(r0, 64)])
            return carry
        lax.fori_loop(0, zrows // 64, _zs, 0)
        plsc.subcore_barrier()

        def chunk_body(k, carry):
            i = wid + k * NW

            @pl.when(i < n_chunks)
            def _():
                base = i * CHUNK
                pltpu.sync_copy(rows_h.at[pl.ds(base, CHUNK)], r_v)
                pltpu.sync_copy(cols_h.at[pl.ds(base, CHUNK)], c_v)
                pltpu.sync_copy(attr_h.at[pl.ds(base, CHUNK)], a_v)

                def grp(g, carry2):
                    r16 = r_v[pl.ds(g * L, L)]
                    c16 = c_v[pl.ds(g * L, L)]
                    m = r16 != c16
                    mf = jnp.where(m, 1.0, 0.0).astype(F32)
                    nmf = (1.0 - mf).astype(F32)
                    cb = jnp.where(m, trash16, c16)
                    j = g // (SUB // L)
                    o = (g % (SUB // L)) * L
                    ia_v[j, pl.ds(o, L)] = c16
                    ib_v[j, pl.ds(o, L)] = cb
                    plsc.addupdate_scatter(cnt_v, [c16], ones16)
                    plsc.addupdate_scatter(slc_v, [c16], nmf)
                    plsc.addupdate_scatter(cra_v, [r16], mf)
                    return carry2
                lax.fori_loop(0, GP, grp, 0)
                for j in range(NSUB):
                    pltpu.sync_copy(a_v.at[pl.ds(j * SUB, SUB)],
                                    sa_s.at[ia_v.at[j]], add=True)
                    pltpu.sync_copy(a_v.at[pl.ds(j * SUB, SUB)],
                                    sl_s.at[ib_v.at[j]], add=True)
            return carry
        lax.fori_loop(0, max_chunks, chunk_body, 0)
        plsc.subcore_barrier()

        r0 = sid * zrows
        pltpu.sync_copy(sa_s.at[pl.ds(r0, zrows)], sa_h.at[cid, pl.ds(r0, zrows)])
        pltpu.sync_copy(sl_s.at[pl.ds(r0, zrows)], sl_h.at[cid, pl.ds(r0, zrows)])
        pltpu.sync_copy(cnt_v, cnt_h.at[0, wid])
        pltpu.sync_copy(slc_v, cnt_h.at[1, wid])
        pltpu.sync_copy(cra_v, cnt_h.at[2, wid])

    return pl.kernel(
        body,
        out_type=(jax.ShapeDtypeStruct((NC, n_pad, 16), F32),
                  jax.ShapeDtypeStruct((NC, n_pad, 16), F32),
                  jax.ShapeDtypeStruct((3, NW, n_pad), F32)),
        mesh=_sc_mesh(),
        scratch_types=[
            pltpu.VMEM((CHUNK,), I32),
            pltpu.VMEM((CHUNK,), I32),
            pltpu.VMEM((CHUNK, 16), F32),
            pltpu.VMEM((NSUB, SUB), I32),
            pltpu.VMEM((NSUB, SUB), I32),
            pltpu.VMEM((n_pad,), F32),
            pltpu.VMEM((n_pad,), F32),
            pltpu.VMEM((n_pad,), F32),
            pltpu.VMEM((64, 16), F32),
            pltpu.VMEM_SHARED((n_pad, 16), F32),
            pltpu.VMEM_SHARED((n_pad, 16), F32),
        ],
        **_SC_PARAMS,
    )


def _make_pass2a(E, n, n_pad):
    CHUNK = 256
    NSUB = CHUNK // SUB
    GP = CHUNK // L
    n_chunks = E // CHUNK
    max_chunks = -(-n_chunks // NW)
    zrows = n_pad // NS

    def body(rows_h, cols_h, y_h,
             acc_h,
             r_v, c_v, ir_v, ic_v, y_v, z_v,
             acc_s, sem):
        cid = lax.axis_index("c")
        sid = lax.axis_index("s")
        wid = cid * NS + sid
        zero16 = jnp.zeros((L,), F32)
        trash16 = jnp.full((L,), n, I32)

        def _zz(i, carry):
            for q in range(128 // L):
                z_v[i, pl.ds(q * L, L)] = zero16
            return carry
        lax.fori_loop(0, 32, _zz, 0)

        def _zs(i, carry):
            pltpu.sync_copy(z_v, acc_s.at[pl.ds(sid * zrows + i * 32, 32)])
            return carry
        lax.fori_loop(0, zrows // 32, _zs, 0)
        plsc.subcore_barrier()

        def chunk_body(k, carry):
            i = wid + k * NW

            @pl.when(i < n_chunks)
            def _():
                base = i * CHUNK
                pltpu.sync_copy(rows_h.at[pl.ds(base, CHUNK)], r_v)
                pltpu.sync_copy(cols_h.at[pl.ds(base, CHUNK)], c_v)

                def grp(g, carry2):
                    r16 = r_v[pl.ds(g * L, L)]
                    c16 = c_v[pl.ds(g * L, L)]
                    csc = jnp.where(r16 != c16, c16, trash16)
                    j = g // (SUB // L)
                    o = (g % (SUB // L)) * L
                    ir_v[j, pl.ds(o, L)] = r16
                    ic_v[j, pl.ds(o, L)] = csc
                    return carry2
                lax.fori_loop(0, GP, grp, 0)

                descs = []
                for j in range(NSUB):
                    descs.append(pltpu.async_copy(
                        y_h.at[ir_v.at[j]],
                        y_v.at[pl.ds(j * SUB, SUB)], sem))
                for d in descs:
                    d.wait()
                for j in range(NSUB):
                    pltpu.sync_copy(y_v.at[pl.ds(j * SUB, SUB)],
                                    acc_s.at[ic_v.at[j]], add=True)
            return carry
        lax.fori_loop(0, max_chunks, chunk_body, 0)
        plsc.subcore_barrier()

        r0 = sid * zrows
        pltpu.sync_copy(acc_s.at[pl.ds(r0, zrows)],
                        acc_h.at[cid, pl.ds(r0, zrows)])

    return pl.kernel(
        body,
        out_type=jax.ShapeDtypeStruct((NC, n_pad, 128), F32),
        mesh=_sc_mesh(),
        scratch_types=[
            pltpu.VMEM((CHUNK,), I32),
            pltpu.VMEM((CHUNK,), I32),
            pltpu.VMEM((NSUB, SUB), I32),
            pltpu.VMEM((NSUB, SUB), I32),
            pltpu.VMEM((CHUNK, 128), F32),
            pltpu.VMEM((32, 128), F32),
            pltpu.VMEM_SHARED((n_pad, 128), F32),
            pltpu.SemaphoreType.DMA,
        ],
        **_SC_PARAMS,
    )


def _make_pass2b(E, n, n_pad):
    CHUNK = 512
    NSUB = CHUNK // SUB
    GP = CHUNK // L
    n_chunks = E // CHUNK
    max_chunks = -(-n_chunks // NW)
    zrows = n_pad // NS

    def body(rows_h, cols_h, attr_h, t_h, dis2_h,
             acc_h, vs_h,
             r_v, c_v, a_v, ir_v, ic_v, t_v, d2_v, vs_v, z_v,
             acc_s, sem):
        cid = lax.axis_index("c")
        sid = lax.axis_index("s")
        wid = cid * NS + sid
        zero16 = jnp.zeros((L,), F32)
        trash16 = jnp.full((L,), n, I32)

        pltpu.sync_copy(dis2_h, d2_v)

        def _zz(i, carry):
            z_v[i, :] = zero16
            return carry
        lax.fori_loop(0, 64, _zz, 0)

        def _zc(i, carry):
            vs_v[pl.ds(i * L, L)] = zero16
            return carry
        lax.fori_loop(0, n_pad // L, _zc, 0)

        def _zs(i, carry):
            pltpu.sync_copy(z_v, acc_s.at[pl.ds(sid * zrows + i * 64, 64)])
            return carry
        lax.fori_loop(0, zrows // 64, _zs, 0)
        plsc.subcore_barrier()

        def chunk_body(k, carry):
            i = wid + k * NW

            @pl.when(i < n_chunks)
            def _():
                base = i * CHUNK
                pltpu.sync_copy(rows_h.at[pl.ds(base, CHUNK)], r_v)
                pltpu.sync_copy(cols_h.at[pl.ds(base, CHUNK)], c_v)
                pltpu.sync_copy(attr_h.at[pl.ds(base, CHUNK)], a_v)

                def grp(g, carry2):
                    r16 = r_v[pl.ds(g * L, L)]
                    c16 = c_v[pl.ds(g * L, L)]
                    csc = jnp.where(r16 != c16, c16, trash16)
                    j = g // (SUB // L)
                    o = (g % (SUB // L)) * L
                    ir_v[j, pl.ds(o, L)] = r16
                    ic_v[j, pl.ds(o, L)] = csc
                    d2r = plsc.load_gather(d2_v, [r16])
                    plsc.addupdate_scatter(vs_v, [csc], d2r)
                    return carry2
                lax.fori_loop(0, GP, grp, 0)

                descs = []
                for j in range(NSUB):
                    descs.append(pltpu.async_copy(
                        t_h.at[ir_v.at[j]],
                        t_v.at[pl.ds(j * SUB, SUB)], sem))
                for d in descs:
                    d.wait()

                def edge(e, carry2):
                    t_v[e, :] = t_v[e, :] * a_v[e, :]
                    return carry2
                lax.fori_loop(0, CHUNK, edge, 0)

                for j in range(NSUB):
                    pltpu.sync_copy(t_v.at[pl.ds(j * SUB, SUB)],
                                    acc_s.at[ic_v.at[j]], add=True)
            return carry
        lax.fori_loop(0, max_chunks, chunk_body, 0)
        plsc.subcore_barrier()

        r0 = sid * zrows
        pltpu.sync_copy(acc_s.at[pl.ds(r0, zrows)],
                        acc_h.at[cid, pl.ds(r0, zrows)])
        pltpu.sync_copy(vs_v, vs_h.at[wid])

    return pl.kernel(
        body,
        out_type=(jax.ShapeDtypeStruct((NC, n_pad, 16), F32),
                  jax.ShapeDtypeStruct((NW, n_pad), F32)),
        mesh=_sc_mesh(),
        scratch_types=[
            pltpu.VMEM((CHUNK,), I32),
            pltpu.VMEM((CHUNK,), I32),
            pltpu.VMEM((CHUNK, 16), F32),
            pltpu.VMEM((NSUB, SUB), I32),
            pltpu.VMEM((NSUB, SUB), I32),
            pltpu.VMEM((CHUNK, 16), F32),
            pltpu.VMEM((n_pad,), F32),
            pltpu.VMEM((n_pad,), F32),
            pltpu.VMEM((64, 16), F32),
            pltpu.VMEM_SHARED((n_pad, 16), F32),
            pltpu.SemaphoreType.DMA,
        ],
        **_SC_PARAMS,
    )


def _make_tcmid(n_pad):
    BLK = 512
    grid = (n_pad // BLK,)

    def body(sa_ref, sl_ref, cnt_ref, x_ref, y_ref, t_ref, dis2_ref, aux_ref):
        sa = sa_ref[0] + sa_ref[1]
        sl = sl_ref[0] + sl_ref[1]
        cnt = jnp.sum(cnt_ref[...], axis=1)
        cnt_t = jnp.transpose(cnt, (1, 0))
        cnt_all = cnt_t[:, 0:1]
        slc = cnt_t[:, 1:2]
        cra = cnt_t[:, 2:3]
        la_mean = sa / jnp.maximum(cnt_all, 1.0)
        la_sl = sl / jnp.maximum(slc, 1.0)
        flag = jnp.minimum(slc, 1.0)
        la = flag * la_sl + (1.0 - flag) * la_mean
        degw = sa - sl + la
        dis = jnp.where(degw > 0.0, lax.rsqrt(jnp.maximum(degw, 1e-30)), 0.0)
        dis2 = lax.rsqrt(cra + 1.0)
        y_ref[...] = dis2 * x_ref[...]
        t_ref[...] = dis2 * dis
        dis2_ref[...] = lax.rsqrt(cnt[2:3, :] + 1.0)
        aux_ref[:, :16] = la
        aux_ref[:, 16:] = jnp.broadcast_to(dis2, (BLK, 16))

    return pl.pallas_call(
        body,
        grid=grid,
        in_specs=[
            pl.BlockSpec((NC, BLK, 16), lambda i: (0, i, 0)),
            pl.BlockSpec((NC, BLK, 16), lambda i: (0, i, 0)),
            pl.BlockSpec((3, NW, BLK), lambda i: (0, 0, i)),
            pl.BlockSpec((BLK, 128), lambda i: (i, 0)),
        ],
        out_specs=[
            pl.BlockSpec((BLK, 128), lambda i: (i, 0)),
            pl.BlockSpec((BLK, 16), lambda i: (i, 0)),
            pl.BlockSpec((1, BLK), lambda i: (0, i)),
            pl.BlockSpec((BLK, 32), lambda i: (i, 0)),
        ],
        out_shape=(jax.ShapeDtypeStruct((n_pad, 128), F32),
                   jax.ShapeDtypeStruct((n_pad, 16), F32),
                   jax.ShapeDtypeStruct((1, n_pad), F32),
                   jax.ShapeDtypeStruct((n_pad, 32), F32)),
    )


def _make_tcpost(n_pad):
    BLK = 512
    grid = (n_pad // BLK,)

    def body(acc_ref, acc16_ref, vs_ref, y_ref, t_ref, aux_ref,
             wet_ref, be_ref, wt_ref, bias_ref, out_ref):
        u = acc_ref[0] + acc_ref[1]
        v16 = acc16_ref[0] + acc16_ref[1]
        vs = jnp.sum(vs_ref[...], axis=0, keepdims=True)
        vs_c = jnp.transpose(vs, (1, 0))
        y = y_ref[...]
        t = t_ref[...]
        la = aux_ref[:, :16]
        dis2 = aux_ref[:, 16:17]
        a = dis2 * (u + y)
        b = t * (v16 + t * la)
        s = dis2 * (vs_c + dis2)
        pre = (a + jnp.dot(b, wet_ref[...], preferred_element_type=F32)
               + s * be_ref[...])
        out_ref[...] = (jnp.dot(pre, wt_ref[...], preferred_element_type=F32)
                        + bias_ref[...])

    return pl.pallas_call(
        body,
        grid=grid,
        in_specs=[
            pl.BlockSpec((NC, BLK, 128), lambda i: (0, i, 0)),
            pl.BlockSpec((NC, BLK, 16), lambda i: (0, i, 0)),
            pl.BlockSpec((NW, BLK), lambda i: (0, i)),
            pl.BlockSpec((BLK, 128), lambda i: (i, 0)),
            pl.BlockSpec((BLK, 16), lambda i: (i, 0)),
            pl.BlockSpec((BLK, 32), lambda i: (i, 0)),
            pl.BlockSpec((16, 128), lambda i: (0, 0)),
            pl.BlockSpec((1, 128), lambda i: (0, 0)),
            pl.BlockSpec((128, 128), lambda i: (0, 0)),
            pl.BlockSpec((1, 128), lambda i: (0, 0)),
        ],
        out_specs=pl.BlockSpec((BLK, 128), lambda i: (i, 0)),
        out_shape=jax.ShapeDtypeStruct((n_pad, 128), F32),
    )


def kernel(x, edge_index, edge_attr, W, We, be, bias):
    n, d_in = x.shape
    E = edge_index.shape[1]
    n_pad = -(-(n + 1) // 1024) * 1024

    rows = edge_index[0].astype(I32)
    cols = edge_index[1].astype(I32)
    attr = edge_attr.astype(F32)
    x_pad = jnp.pad(x.astype(F32), ((0, n_pad - n), (0, 0)))

    sa, sl, cnt = _make_pass1(E, n, n_pad)(rows, cols, attr)
    y, t, dis2t, aux = _make_tcmid(n_pad)(sa, sl, cnt, x_pad)
    acc128 = _make_pass2a(E, n, n_pad)(rows, cols, y)
    acc16, vs = _make_pass2b(E, n, n_pad)(rows, cols, attr, t, dis2t[0])
    out = _make_tcpost(n_pad)(acc128, acc16, vs, y, t, aux,
                              We.T.astype(F32), be.astype(F32)[None],
                              W.T.astype(F32), bias.astype(F32)[None])
    return out[:n]
